# Initial kernel scaffold; baseline (speedup 1.0000x reference)
import jax
import jax.numpy as jnp
from jax.experimental import pallas as pl


def kernel(x, edge_index, batch, W1, b1, W2, b2, cw1, cb1, cw2, cb2, cw3, cb3, fw1, fb1, fw2, fb2):
    raise NotImplementedError("write your pallas kernel here")


# speedup vs baseline: 19.5340x; 19.5340x over previous
import functools

import jax
import jax.numpy as jnp
from jax import lax
from jax.experimental import pallas as pl
from jax.experimental.pallas import tpu as pltpu
from jax.experimental.pallas import tpu_sc as plsc

N = 10000
NPAD = 10240
D = 16
E = 320000
NG = 10
LMAX = 2000
LBLK = 2048
NPARTS = 10

GROUP = 128
BLK = 16
GPT = 80
NTILES = 32
EPAD = NTILES * GPT * GROUP
EROWS = EPAD // GROUP
ZSL = NPAD // 16

DDUMP = NG * LBLK
DROWS = 20608
DSL = DROWS // 16
NPT = NPAD // 16

_f32 = jnp.float32


_MESH = dict(core_axis_name="c", subcore_axis_name="s")
_SC_PARAMS = pltpu.CompilerParams(use_tc_tiling_on_sc=False)


def _sc_hist(dstp, zeros_nd, ones_rows):
    @functools.partial(
        pl.kernel,
        out_type=jax.ShapeDtypeStruct((2, NPAD, D), _f32),
        mesh=plsc.VectorSubcoreMesh(**_MESH),
        compiler_params=_SC_PARAMS,
        scratch_types=[
            pltpu.VMEM((BLK, GROUP), jnp.int32),
            pltpu.VMEM((GROUP, D), _f32),
            pltpu.VMEM_SHARED((NPAD, D), _f32),
        ],
    )
    def k(dst_hbm, z_hbm, ones_hbm, out_hbm, didx, ones_v, accum):
        c = lax.axis_index("c")
        s = lax.axis_index("s")
        t = s * 2 + c
        pltpu.sync_copy(z_hbm.at[pl.ds(s * ZSL, ZSL)], accum.at[pl.ds(s * ZSL, ZSL)])
        pltpu.sync_copy(ones_hbm, ones_v)
        plsc.subcore_barrier()

        @pl.loop(0, GPT // BLK)
        def _blk(blk):
            r0 = t * GPT + blk * BLK
            pltpu.sync_copy(dst_hbm.at[pl.ds(r0, BLK)], didx)

            @pl.loop(0, BLK)
            def _j(j):
                pltpu.sync_copy(ones_v, accum.at[didx.at[j]], add=True)

        plsc.subcore_barrier()
        pltpu.sync_copy(accum.at[pl.ds(s * ZSL, ZSL)],
                        out_hbm.at[c, pl.ds(s * ZSL, ZSL)])

    return k(dstp, zeros_nd, ones_rows)


def _sc_agg(g, srcp, dstp, zeros_nd):
    @functools.partial(
        pl.kernel,
        out_type=jax.ShapeDtypeStruct((2, NPAD, D), _f32),
        mesh=plsc.VectorSubcoreMesh(**_MESH),
        compiler_params=_SC_PARAMS,
        scratch_types=[
            pltpu.VMEM((BLK, GROUP), jnp.int32),
            pltpu.VMEM((BLK, GROUP), jnp.int32),
            pltpu.VMEM((GROUP, D), _f32),
            pltpu.VMEM_SHARED((NPAD, D), _f32),
        ],
    )
    def k(g_hbm, src_hbm, dst_hbm, z_hbm, out_hbm, sidx, didx, rows_v, accum):
        c = lax.axis_index("c")
        s = lax.axis_index("s")
        t = s * 2 + c
        pltpu.sync_copy(z_hbm.at[pl.ds(s * ZSL, ZSL)], accum.at[pl.ds(s * ZSL, ZSL)])
        plsc.subcore_barrier()

        @pl.loop(0, GPT // BLK)
        def _blk(blk):
            r0 = t * GPT + blk * BLK
            pltpu.sync_copy(src_hbm.at[pl.ds(r0, BLK)], sidx)
            pltpu.sync_copy(dst_hbm.at[pl.ds(r0, BLK)], didx)

            @pl.loop(0, BLK)
            def _j(j):
                pltpu.sync_copy(g_hbm.at[sidx.at[j]], rows_v)
                pltpu.sync_copy(rows_v, accum.at[didx.at[j]], add=True)

        plsc.subcore_barrier()
        pltpu.sync_copy(accum.at[pl.ds(s * ZSL, ZSL)],
                        out_hbm.at[c, pl.ds(s * ZSL, ZSL)])

    return k(g, srcp, dstp, zeros_nd)


def _sc_dense(h2f, slotsr, zeros_dense):
    @functools.partial(
        pl.kernel,
        out_type=jax.ShapeDtypeStruct((DROWS, D), _f32),
        mesh=plsc.VectorSubcoreMesh(**_MESH),
        compiler_params=_SC_PARAMS,
        scratch_types=[
            pltpu.VMEM((NPT // GROUP, GROUP), jnp.int32),
            pltpu.VMEM((GROUP, D), _f32),
            pltpu.VMEM_SHARED((DROWS, D), _f32),
        ],
    )
    def k(h_hbm, slot_hbm, z_hbm, out_hbm, sidx, rows_v, densebuf):
        c = lax.axis_index("c")
        s = lax.axis_index("s")

        @pl.when(c == 0)
        def _():
            pltpu.sync_copy(z_hbm.at[pl.ds(s * DSL, DSL)],
                            densebuf.at[pl.ds(s * DSL, DSL)])
            pltpu.sync_copy(slot_hbm.at[pl.ds(s * (NPT // GROUP), NPT // GROUP)],
                            sidx)
            plsc.subcore_barrier()

            @pl.loop(0, NPT // GROUP)
            def _j(j):
                pltpu.sync_copy(h_hbm.at[pl.ds(s * NPT + j * GROUP, GROUP)],
                                rows_v)
                pltpu.sync_copy(rows_v, densebuf.at[sidx.at[j]])

            plsc.subcore_barrier()
            pltpu.sync_copy(densebuf.at[pl.ds(s * DSL, DSL)],
                            out_hbm.at[pl.ds(s * DSL, DSL)])

    return k(h2f, slotsr, zeros_dense)


def _mm1(xp, W1p):
    def body(x_ref, w_ref, o_ref):
        o_ref[...] = jnp.dot(x_ref[...], w_ref[...],
                             preferred_element_type=_f32)

    return pl.pallas_call(
        body, out_shape=jax.ShapeDtypeStruct((NPAD, D), _f32))(xp, W1p)


def _prep(hist, h1, batpad):
    def body(hp_ref, h_ref, bat_ref, dinv_ref, g_ref, slot_ref):
        deg = 1.0 + hp_ref[0, :, 0:1] + hp_ref[1, :, 0:1]
        dinv = lax.rsqrt(deg)
        dinv_ref[...] = dinv
        g_ref[...] = dinv * h_ref[...]

        bat = bat_ref[...]
        off = jnp.zeros((1, NPAD), jnp.int32)
        cum = jnp.zeros((), jnp.int32)
        for b in range(NG):
            off = jnp.where(bat == b, b * LBLK - cum, off)
            cum = cum + jnp.sum((bat == b).astype(jnp.int32))
        idx = lax.broadcasted_iota(jnp.int32, (1, NPAD), 1)
        slot_ref[...] = jnp.where(bat < NG, idx + off, DDUMP)

    return pl.pallas_call(
        body,
        out_shape=(jax.ShapeDtypeStruct((NPAD, 1), _f32),
                   jax.ShapeDtypeStruct((NPAD, D), _f32),
                   jax.ShapeDtypeStruct((1, NPAD), jnp.int32)))(
            hist, h1, batpad)


def _mid(p, g1, dinv, b1p, W2p):
    def body(p_ref, g_ref, di_ref, b_ref, w_ref, o_ref):
        agg = p_ref[0] + p_ref[1] + g_ref[...]
        h1f = jnp.maximum(di_ref[...] * agg + b_ref[...], 0.0)
        h2 = jnp.dot(h1f, w_ref[...], preferred_element_type=_f32)
        o_ref[...] = di_ref[...] * h2

    return pl.pallas_call(
        body, out_shape=jax.ShapeDtypeStruct((NPAD, D), _f32))(
            p, g1, dinv, b1p, W2p)


def _finalize(p, g2, dinv, b2p):
    def body(p_ref, g_ref, di_ref, b_ref, o_ref):
        agg = p_ref[0] + p_ref[1] + g_ref[...]
        o_ref[...] = jnp.maximum(di_ref[...] * agg + b_ref[...], 0.0)

    return pl.pallas_call(
        body, out_shape=jax.ShapeDtypeStruct((NPAD, D), _f32))(
            p, g2, dinv, b2p)


def _conv_block(X, w, bias, P, L):
    Cin = X.shape[1]
    z = jnp.zeros((2, Cin), _f32)
    Xp = jnp.concatenate([z, X, z], axis=0)
    s = None
    for k in range(5):
        term = jnp.dot(Xp[k:k + L], w[k], preferred_element_type=_f32)
        s = term if s is None else s + term
    Y = jnp.maximum(s + bias, 0.0)
    return jnp.dot(P, Y, preferred_element_type=_f32)


def _tail(dense, batch2d, W1bd, W2bd, W3bd, cb1t, cb2t, cb3t,
          P1, P2, P3, fw1rt, fb1r, fw2t, fb2r):
    def body(d_ref, bat_ref, w1_ref, w2_ref, w3_ref,
             c1_ref, c2_ref, c3_ref, p1_ref, p2_ref, p3_ref,
             f1_ref, fb1_ref, f2_ref, fb2_ref, o_ref):
        X = jnp.concatenate(
            [d_ref[b * LBLK:b * LBLK + LMAX, :] for b in range(NG)],
            axis=1)

        X = _conv_block(X, w1_ref[...], c1_ref[...], p1_ref[...], LMAX)
        X = _conv_block(X, w2_ref[...], c2_ref[...], p2_ref[...], LMAX // 2)
        X = _conv_block(X, w3_ref[...], c3_ref[...], p3_ref[...], LMAX // 4)

        bat = bat_ref[...]
        nns = [jnp.sum((bat == b).astype(jnp.int32)) for b in range(NG)]

        L3 = LMAX // 8
        pos = lax.broadcasted_iota(jnp.int32, (NPARTS, L3), 1)
        jcol = lax.broadcasted_iota(jnp.int32, (NPARTS, 1), 0)
        means_list = []
        for b in range(NG):
            Xb = X[:, b * 64:(b + 1) * 64]
            valid = nns[b] // 8
            base = valid // NPARTS
            rem = valid % NPARTS
            szj = base + (jcol < rem).astype(jnp.int32)
            startj = jcol * base + jnp.minimum(jcol, rem)
            mask = ((pos >= startj) & (pos < startj + szj)).astype(_f32)
            sums = jnp.dot(mask, Xb, preferred_element_type=_f32)
            means_list.append(sums / szj.astype(_f32))

        f1 = f1_ref[...]
        acc = jnp.zeros((NG, 100), _f32)
        for j in range(NPARTS):
            Mj = jnp.concatenate(
                [means_list[b][j:j + 1, :] for b in range(NG)], axis=0)
            acc = acc + jnp.dot(Mj, f1[j * 64:(j + 1) * 64, :],
                                preferred_element_type=_f32)
        hid = jnp.maximum(acc + fb1_ref[...], 0.0)
        o_ref[...] = (jnp.dot(hid, f2_ref[...], preferred_element_type=_f32)
                      + fb2_ref[...])

    return pl.pallas_call(
        body, out_shape=jax.ShapeDtypeStruct((NG, 2), _f32))(
            dense, batch2d, W1bd, W2bd, W3bd, cb1t, cb2t, cb3t,
            P1, P2, P3, fw1rt, fb1r, fw2t, fb2r)


def _block_diag(wk, B):
    K, ci, co = wk.shape
    out = jnp.zeros((K, ci * B, co * B), wk.dtype)
    for b in range(B):
        out = out.at[:, b * ci:(b + 1) * ci, b * co:(b + 1) * co].set(wk)
    return out


def _pool_mat(L):
    r2 = jnp.arange(L // 2, dtype=jnp.int32)[:, None] * 2
    c = jnp.arange(L, dtype=jnp.int32)[None, :]
    return jnp.where((c == r2) | (c == r2 + 1), 0.5, 0.0).astype(_f32)


def kernel(x, edge_index, batch, W1, b1, W2, b2, cw1, cb1, cw2, cb2,
           cw3, cb3, fw1, fb1, fw2, fb2):
    src = edge_index[0]
    dst = edge_index[1]
    pad = jnp.full((EPAD - E,), N, dtype=src.dtype)
    srcp = jnp.concatenate([src, pad]).reshape(EROWS, GROUP)
    dstp = jnp.concatenate([dst, pad]).reshape(EROWS, GROUP)

    xp = jnp.pad(x, ((0, NPAD - N), (0, 0)))
    W1p = jnp.pad(W1, ((0, 0), (0, D - W1.shape[1])))
    W2p = jnp.pad(W2, ((0, D - 8), (0, D - 8)))
    b1p = jnp.pad(b1, (0, D - 8)).reshape(1, D)
    b2p = jnp.pad(b2, (0, D - 8)).reshape(1, D)
    zeros_nd = jnp.zeros((NPAD, D), _f32)
    zeros_dense = jnp.zeros((DROWS, D), _f32)
    ones_rows = jnp.zeros((GROUP, D), _f32).at[:, 0].set(1.0)
    batch2d = batch.reshape(1, N)
    batpad = jnp.pad(batch2d, ((0, 0), (0, NPAD - N)), constant_values=NG)

    wk1 = jnp.pad(jnp.transpose(cw1, (2, 1, 0)), ((0, 0), (0, 8), (0, 0)))
    W1bd = _block_diag(wk1, NG)
    W2bd = _block_diag(jnp.transpose(cw2, (2, 1, 0)), NG)
    W3bd = _block_diag(jnp.transpose(cw3, (2, 1, 0)), NG)
    cb1t = jnp.tile(cb1, NG).reshape(1, -1)
    cb2t = jnp.tile(cb2, NG).reshape(1, -1)
    cb3t = jnp.tile(cb3, NG).reshape(1, -1)
    P1 = _pool_mat(LMAX)
    P2 = _pool_mat(LMAX // 2)
    P3 = _pool_mat(LMAX // 4)
    fw1rt = fw1.reshape(100, 64, NPARTS).transpose(0, 2, 1).reshape(100, -1).T
    fb1r = fb1.reshape(1, -1)
    fw2t = fw2.T
    fb2r = fb2.reshape(1, -1)

    hist = _sc_hist(dstp, zeros_nd, ones_rows)
    h1 = _mm1(xp, W1p)
    dinv, g1, slots = _prep(hist, h1, batpad)
    slotsr = slots.reshape(NPAD // GROUP, GROUP)
    p1 = _sc_agg(g1, srcp, dstp, zeros_nd)
    g2 = _mid(p1, g1, dinv, b1p, W2p)
    p2 = _sc_agg(g2, srcp, dstp, zeros_nd)
    h2f = _finalize(p2, g2, dinv, b2p)
    dense = _sc_dense(h2f, slotsr, zeros_dense)
    return _tail(dense, batch2d, W1bd, W2bd, W3bd, cb1t, cb2t, cb3t,
                 P1, P2, P3, fw1rt, fb1r, fw2t, fb2r)


# speedup vs baseline: 23.6091x; 1.2086x over previous
import functools

import jax
import jax.numpy as jnp
from jax import lax
from jax.experimental import pallas as pl
from jax.experimental.pallas import tpu as pltpu
from jax.experimental.pallas import tpu_sc as plsc

N = 10000
NPAD = 10240
D = 16
E = 320000
NG = 10
LMAX = 2000
LBLK = 2048
NPARTS = 10

GROUP = 128
BLK = 16
GPT = 80
NTILES = 32
EPAD = NTILES * GPT * GROUP
EROWS = EPAD // GROUP
ZSL = NPAD // 16

DDUMP = NG * LBLK
DROWS = 20608
DSL = DROWS // 16
NPT = NPAD // 16

_f32 = jnp.float32


_MESH = dict(core_axis_name="c", subcore_axis_name="s")
_SC_PARAMS = pltpu.CompilerParams(use_tc_tiling_on_sc=False)


def _sc_hist(dstp, zeros_nd, ones_rows):
    @functools.partial(
        pl.kernel,
        out_type=jax.ShapeDtypeStruct((2, NPAD, D), _f32),
        mesh=plsc.VectorSubcoreMesh(**_MESH),
        compiler_params=_SC_PARAMS,
        scratch_types=[
            pltpu.VMEM((BLK, GROUP), jnp.int32),
            pltpu.VMEM((GROUP, D), _f32),
            pltpu.VMEM_SHARED((NPAD, D), _f32),
        ],
    )
    def k(dst_hbm, z_hbm, ones_hbm, out_hbm, didx, ones_v, accum):
        c = lax.axis_index("c")
        s = lax.axis_index("s")
        t = s * 2 + c
        pltpu.sync_copy(z_hbm.at[pl.ds(s * ZSL, ZSL)], accum.at[pl.ds(s * ZSL, ZSL)])
        pltpu.sync_copy(ones_hbm, ones_v)
        plsc.subcore_barrier()

        @pl.loop(0, GPT // BLK)
        def _blk(blk):
            r0 = t * GPT + blk * BLK
            pltpu.sync_copy(dst_hbm.at[pl.ds(r0, BLK)], didx)

            @pl.loop(0, BLK)
            def _j(j):
                pltpu.sync_copy(ones_v, accum.at[didx.at[j]], add=True)

        plsc.subcore_barrier()
        pltpu.sync_copy(accum.at[pl.ds(s * ZSL, ZSL)],
                        out_hbm.at[c, pl.ds(s * ZSL, ZSL)])

    return k(dstp, zeros_nd, ones_rows)


ABLK = 20
ANB = GPT // ABLK


def _sc_agg(g, srcp, dstp, zeros_nd):
    @functools.partial(
        pl.kernel,
        out_type=jax.ShapeDtypeStruct((2, NPAD, D), _f32),
        mesh=plsc.VectorSubcoreMesh(**_MESH),
        compiler_params=_SC_PARAMS,
        scratch_types=[
            pltpu.VMEM((ABLK, GROUP), jnp.int32),
            pltpu.VMEM((ABLK, GROUP), jnp.int32),
            pltpu.VMEM((ABLK, GROUP), jnp.int32),
            pltpu.VMEM((ABLK, GROUP), jnp.int32),
            pltpu.VMEM((ABLK, GROUP, D), _f32),
            pltpu.VMEM((ABLK, GROUP, D), _f32),
            pltpu.SemaphoreType.DMA,
            pltpu.SemaphoreType.DMA,
            pltpu.VMEM_SHARED((NPAD, D), _f32),
        ],
    )
    def k(g_hbm, src_hbm, dst_hbm, z_hbm, out_hbm,
          sidx0, didx0, sidx1, didx1, rows0, rows1, sem0, sem1, accum):
        c = lax.axis_index("c")
        s = lax.axis_index("s")
        t = s * 2 + c
        pltpu.sync_copy(z_hbm.at[pl.ds(s * ZSL, ZSL)], accum.at[pl.ds(s * ZSL, ZSL)])
        plsc.subcore_barrier()

        bufs = ((sidx0, didx0, rows0, sem0), (sidx1, didx1, rows1, sem1))
        gds = [None, None]

        def stage(p, blk):
            sidx, didx, rows, sem = bufs[p]
            r0 = t * GPT + blk * ABLK
            pltpu.sync_copy(src_hbm.at[pl.ds(r0, ABLK)], sidx)
            pltpu.sync_copy(dst_hbm.at[pl.ds(r0, ABLK)], didx)
            gds[p] = [pltpu.async_copy(g_hbm.at[sidx.at[j]], rows.at[j], sem)
                      for j in range(ABLK)]

        def process(p):
            sidx, didx, rows, sem = bufs[p]
            for d in gds[p]:
                d.wait()
            for j in range(ABLK):
                pltpu.sync_copy(rows.at[j], accum.at[didx.at[j]], add=True)

        stage(0, 0)
        for blk in range(ANB):
            if blk + 1 < ANB:
                stage((blk + 1) % 2, blk + 1)
            process(blk % 2)

        plsc.subcore_barrier()
        pltpu.sync_copy(accum.at[pl.ds(s * ZSL, ZSL)],
                        out_hbm.at[c, pl.ds(s * ZSL, ZSL)])

    return k(g, srcp, dstp, zeros_nd)


def _sc_dense(h2f, slotsr, zeros_dense):
    @functools.partial(
        pl.kernel,
        out_type=jax.ShapeDtypeStruct((DROWS, D), _f32),
        mesh=plsc.VectorSubcoreMesh(**_MESH),
        compiler_params=_SC_PARAMS,
        scratch_types=[
            pltpu.VMEM((NPT // GROUP, GROUP), jnp.int32),
            pltpu.VMEM((GROUP, D), _f32),
            pltpu.VMEM_SHARED((DROWS, D), _f32),
        ],
    )
    def k(h_hbm, slot_hbm, z_hbm, out_hbm, sidx, rows_v, densebuf):
        c = lax.axis_index("c")
        s = lax.axis_index("s")

        @pl.when(c == 0)
        def _():
            pltpu.sync_copy(z_hbm.at[pl.ds(s * DSL, DSL)],
                            densebuf.at[pl.ds(s * DSL, DSL)])
            pltpu.sync_copy(slot_hbm.at[pl.ds(s * (NPT // GROUP), NPT // GROUP)],
                            sidx)
            plsc.subcore_barrier()

            @pl.loop(0, NPT // GROUP)
            def _j(j):
                pltpu.sync_copy(h_hbm.at[pl.ds(s * NPT + j * GROUP, GROUP)],
                                rows_v)
                pltpu.sync_copy(rows_v, densebuf.at[sidx.at[j]])

            plsc.subcore_barrier()
            pltpu.sync_copy(densebuf.at[pl.ds(s * DSL, DSL)],
                            out_hbm.at[pl.ds(s * DSL, DSL)])

    return k(h2f, slotsr, zeros_dense)


def _mm1(xp, W1p):
    def body(x_ref, w_ref, o_ref):
        o_ref[...] = jnp.dot(x_ref[...], w_ref[...],
                             preferred_element_type=_f32)

    return pl.pallas_call(
        body, out_shape=jax.ShapeDtypeStruct((NPAD, D), _f32))(xp, W1p)


def _prep(hist, h1, batpad):
    def body(hp_ref, h_ref, bat_ref, dinv_ref, g_ref, slot_ref):
        deg = 1.0 + hp_ref[0, :, 0:1] + hp_ref[1, :, 0:1]
        dinv = lax.rsqrt(deg)
        dinv_ref[...] = dinv
        g_ref[...] = dinv * h_ref[...]

        bat = bat_ref[...]
        off = jnp.zeros((1, NPAD), jnp.int32)
        cum = jnp.zeros((), jnp.int32)
        for b in range(NG):
            off = jnp.where(bat == b, b * LBLK - cum, off)
            cum = cum + jnp.sum((bat == b).astype(jnp.int32))
        idx = lax.broadcasted_iota(jnp.int32, (1, NPAD), 1)
        slot_ref[...] = jnp.where(bat < NG, idx + off, DDUMP)

    return pl.pallas_call(
        body,
        out_shape=(jax.ShapeDtypeStruct((NPAD, 1), _f32),
                   jax.ShapeDtypeStruct((NPAD, D), _f32),
                   jax.ShapeDtypeStruct((1, NPAD), jnp.int32)))(
            hist, h1, batpad)


def _mid(p, g1, dinv, b1p, W2p):
    def body(p_ref, g_ref, di_ref, b_ref, w_ref, o_ref):
        agg = p_ref[0] + p_ref[1] + g_ref[...]
        h1f = jnp.maximum(di_ref[...] * agg + b_ref[...], 0.0)
        h2 = jnp.dot(h1f, w_ref[...], preferred_element_type=_f32)
        o_ref[...] = di_ref[...] * h2

    return pl.pallas_call(
        body, out_shape=jax.ShapeDtypeStruct((NPAD, D), _f32))(
            p, g1, dinv, b1p, W2p)


def _finalize(p, g2, dinv, b2p):
    def body(p_ref, g_ref, di_ref, b_ref, o_ref):
        agg = p_ref[0] + p_ref[1] + g_ref[...]
        o_ref[...] = jnp.maximum(di_ref[...] * agg + b_ref[...], 0.0)

    return pl.pallas_call(
        body, out_shape=jax.ShapeDtypeStruct((NPAD, D), _f32))(
            p, g2, dinv, b2p)


def _conv_block(X, w, bias, P, L):
    Cin = X.shape[1]
    z = jnp.zeros((2, Cin), _f32)
    Xp = jnp.concatenate([z, X, z], axis=0)
    s = None
    for k in range(5):
        term = jnp.dot(Xp[k:k + L], w[k], preferred_element_type=_f32)
        s = term if s is None else s + term
    Y = jnp.maximum(s + bias, 0.0)
    return jnp.dot(P, Y, preferred_element_type=_f32)


def _tail(dense, batch2d, W1bd, W2bd, W3bd, cb1t, cb2t, cb3t,
          P1, P2, P3, fw1rt, fb1r, fw2t, fb2r):
    def body(d_ref, bat_ref, w1_ref, w2_ref, w3_ref,
             c1_ref, c2_ref, c3_ref, p1_ref, p2_ref, p3_ref,
             f1_ref, fb1_ref, f2_ref, fb2_ref, o_ref):
        X = jnp.concatenate(
            [d_ref[b * LBLK:b * LBLK + LMAX, :] for b in range(NG)],
            axis=1)

        X = _conv_block(X, w1_ref[...], c1_ref[...], p1_ref[...], LMAX)
        X = _conv_block(X, w2_ref[...], c2_ref[...], p2_ref[...], LMAX // 2)
        X = _conv_block(X, w3_ref[...], c3_ref[...], p3_ref[...], LMAX // 4)

        bat = bat_ref[...]
        nns = [jnp.sum((bat == b).astype(jnp.int32)) for b in range(NG)]

        L3 = LMAX // 8
        pos = lax.broadcasted_iota(jnp.int32, (NPARTS, L3), 1)
        jcol = lax.broadcasted_iota(jnp.int32, (NPARTS, 1), 0)
        means_list = []
        for b in range(NG):
            Xb = X[:, b * 64:(b + 1) * 64]
            valid = nns[b] // 8
            base = valid // NPARTS
            rem = valid % NPARTS
            szj = base + (jcol < rem).astype(jnp.int32)
            startj = jcol * base + jnp.minimum(jcol, rem)
            mask = ((pos >= startj) & (pos < startj + szj)).astype(_f32)
            sums = jnp.dot(mask, Xb, preferred_element_type=_f32)
            means_list.append(sums / szj.astype(_f32))

        f1 = f1_ref[...]
        acc = jnp.zeros((NG, 100), _f32)
        for j in range(NPARTS):
            Mj = jnp.concatenate(
                [means_list[b][j:j + 1, :] for b in range(NG)], axis=0)
            acc = acc + jnp.dot(Mj, f1[j * 64:(j + 1) * 64, :],
                                preferred_element_type=_f32)
        hid = jnp.maximum(acc + fb1_ref[...], 0.0)
        o_ref[...] = (jnp.dot(hid, f2_ref[...], preferred_element_type=_f32)
                      + fb2_ref[...])

    return pl.pallas_call(
        body, out_shape=jax.ShapeDtypeStruct((NG, 2), _f32))(
            dense, batch2d, W1bd, W2bd, W3bd, cb1t, cb2t, cb3t,
            P1, P2, P3, fw1rt, fb1r, fw2t, fb2r)


def _block_diag(wk, B):
    K, ci, co = wk.shape
    out = jnp.zeros((K, ci * B, co * B), wk.dtype)
    for b in range(B):
        out = out.at[:, b * ci:(b + 1) * ci, b * co:(b + 1) * co].set(wk)
    return out


def _pool_mat(L):
    r2 = jnp.arange(L // 2, dtype=jnp.int32)[:, None] * 2
    c = jnp.arange(L, dtype=jnp.int32)[None, :]
    return jnp.where((c == r2) | (c == r2 + 1), 0.5, 0.0).astype(_f32)


def kernel(x, edge_index, batch, W1, b1, W2, b2, cw1, cb1, cw2, cb2,
           cw3, cb3, fw1, fb1, fw2, fb2):
    src = edge_index[0]
    dst = edge_index[1]
    pad = jnp.full((EPAD - E,), N, dtype=src.dtype)
    srcp = jnp.concatenate([src, pad]).reshape(EROWS, GROUP)
    dstp = jnp.concatenate([dst, pad]).reshape(EROWS, GROUP)

    xp = jnp.pad(x, ((0, NPAD - N), (0, 0)))
    W1p = jnp.pad(W1, ((0, 0), (0, D - W1.shape[1])))
    W2p = jnp.pad(W2, ((0, D - 8), (0, D - 8)))
    b1p = jnp.pad(b1, (0, D - 8)).reshape(1, D)
    b2p = jnp.pad(b2, (0, D - 8)).reshape(1, D)
    zeros_nd = jnp.zeros((NPAD, D), _f32)
    zeros_dense = jnp.zeros((DROWS, D), _f32)
    ones_rows = jnp.zeros((GROUP, D), _f32).at[:, 0].set(1.0)
    batch2d = batch.reshape(1, N)
    batpad = jnp.pad(batch2d, ((0, 0), (0, NPAD - N)), constant_values=NG)

    wk1 = jnp.pad(jnp.transpose(cw1, (2, 1, 0)), ((0, 0), (0, 8), (0, 0)))
    W1bd = _block_diag(wk1, NG)
    W2bd = _block_diag(jnp.transpose(cw2, (2, 1, 0)), NG)
    W3bd = _block_diag(jnp.transpose(cw3, (2, 1, 0)), NG)
    cb1t = jnp.tile(cb1, NG).reshape(1, -1)
    cb2t = jnp.tile(cb2, NG).reshape(1, -1)
    cb3t = jnp.tile(cb3, NG).reshape(1, -1)
    P1 = _pool_mat(LMAX)
    P2 = _pool_mat(LMAX // 2)
    P3 = _pool_mat(LMAX // 4)
    fw1rt = fw1.reshape(100, 64, NPARTS).transpose(0, 2, 1).reshape(100, -1).T
    fb1r = fb1.reshape(1, -1)
    fw2t = fw2.T
    fb2r = fb2.reshape(1, -1)

    hist = _sc_hist(dstp, zeros_nd, ones_rows)
    h1 = _mm1(xp, W1p)
    dinv, g1, slots = _prep(hist, h1, batpad)
    slotsr = slots.reshape(NPAD // GROUP, GROUP)
    p1 = _sc_agg(g1, srcp, dstp, zeros_nd)
    g2 = _mid(p1, g1, dinv, b1p, W2p)
    p2 = _sc_agg(g2, srcp, dstp, zeros_nd)
    h2f = _finalize(p2, g2, dinv, b2p)
    dense = _sc_dense(h2f, slotsr, zeros_dense)
    return _tail(dense, batch2d, W1bd, W2bd, W3bd, cb1t, cb2t, cb3t,
                 P1, P2, P3, fw1rt, fb1r, fw2t, fb2r)
